# baseline (speedup 1.0000x reference)
import functools

import jax
import jax.numpy as jnp
from jax import lax
from jax.experimental import pallas as pl
from jax.experimental.pallas import tpu as pltpu
from jax.experimental.pallas import tpu_sc as plsc

_C = 19
_BINS = _C * _C
_BINS_PAD = 368
_L = 16
_PX_PER_TILE = 65536
_CH = 16384
_NCH = _PX_PER_TILE // _CH


def _body(pred_hbm, label_hbm, out_hbm,
          pb0, pb1, lb0, lb1, hist, merged, sh4, bhist, svec, sc4,
          shared_hist, shared_scores, shared_pad,
          sp0, sp1, sl0, sl1):
    c = lax.axis_index("c")
    s = lax.axis_index("s")
    batch = c * 4 + s // 4
    row0 = (s % 4) * (_PX_PER_TILE // 512)

    lane = lax.iota(jnp.int32, _L)
    lane16 = lane * 16
    ones = jnp.ones((_L,), jnp.float32)
    zeros = jnp.zeros((_L,), jnp.float32)

    @plsc.parallel_loop(0, _BINS_PAD * _L // 256, 1)
    def _(k):
        for u in range(16):
            hist[pl.ds(k * 256 + u * 16, _L)] = zeros

    pbufs = (pb0, pb1)
    lbufs = (lb0, lb1)
    psems = (sp0, sp1)
    lsems = (sl0, sl1)

    rows_per_ch = _CH // 512

    def start(ch):
        slot = ch % 2
        r = row0 + ch * rows_per_ch
        cp = pltpu.async_copy(pred_hbm.at[batch, 0, pl.ds(r, rows_per_ch), :],
                              pbufs[slot], psems[slot])
        cl = pltpu.async_copy(label_hbm.at[batch, 0, pl.ds(r, rows_per_ch), :],
                              lbufs[slot], lsems[slot])
        return cp, cl

    pending = {0: start(0)}
    for ch in range(_NCH):
        if ch + 1 < _NCH:
            pending[ch + 1] = start(ch + 1)
        cp, cl = pending.pop(ch)
        cp.wait()
        cl.wait()
        pb = pbufs[ch % 2]
        lb = lbufs[ch % 2]

        def make_chunk_body(pb, lb):
            def chunk_body(j):
                for u in range(32):
                    p = pb[j, pl.ds(u * 16, _L)]
                    l = lb[j, pl.ds(u * 16, _L)]
                    addr = (p * _C + l) * 16 + lane
                    plsc.addupdate_scatter(hist, [addr], ones)
            return chunk_body
        plsc.parallel_loop(0, _CH // 512, 1, unroll=2)(make_chunk_body(pb, lb))

    @plsc.parallel_loop(0, _BINS_PAD // 16, 1)
    def _(k):
        acc = zeros
        for u in range(16):
            acc = acc + plsc.load_gather(hist, [k * 256 + lane16 + u])
        merged[pl.ds(k * 16, _L)] = acc

    pltpu.sync_copy(merged, shared_hist.at[s])
    plsc.subcore_barrier()

    @pl.when(s < 4)
    def _():
        pltpu.sync_copy(shared_hist.at[pl.ds(s * 4, 4)], sh4)

        def sum4_body(k, _):
            d = pl.ds(k * 16, _L)
            bhist[d] = (sh4[0, d] + sh4[1, d]) + (sh4[2, d] + sh4[3, d])
            return 0
        lax.fori_loop(0, _BINS_PAD // 16, sum4_body, 0)

        for half in range(2):
            cvec = jnp.minimum(half * 16 + lane, _C - 1)
            hi = plsc.load_gather(bhist, [cvec * (_C + 1)])
            hp = jnp.zeros((_L,), jnp.float32)
            hl = jnp.zeros((_L,), jnp.float32)
            for v in range(_C):
                hp = hp + plsc.load_gather(bhist, [cvec * _C + v])
                hl = hl + plsc.load_gather(bhist, [v * _C + cvec])
            merged[pl.ds(half * 16, _L)] = (2.0 * hi) / (hp + hl + 1e-10)

    pltpu.sync_copy(merged, shared_scores.at[s])
    plsc.subcore_barrier()

    @pl.when(s == 0)
    def _():
        pltpu.sync_copy(shared_scores.at[pl.ds(0, 4)], sh4)
        for half in range(2):
            d = pl.ds(half * 16, _L)
            svec[d] = (sh4[0, d] + sh4[1, d]) + (sh4[2, d] + sh4[3, d])
        pltpu.sync_copy(svec, out_hbm.at[c])


@functools.partial(
    pl.kernel,
    out_type=jax.ShapeDtypeStruct((2, 32), jnp.float32),
    mesh=plsc.VectorSubcoreMesh(core_axis_name="c", subcore_axis_name="s"),
    compiler_params=pltpu.CompilerParams(needs_layout_passes=False),
    scratch_types=[
        pltpu.VMEM((_CH // 512, 512), jnp.int32),
        pltpu.VMEM((_CH // 512, 512), jnp.int32),
        pltpu.VMEM((_CH // 512, 512), jnp.int32),
        pltpu.VMEM((_CH // 512, 512), jnp.int32),
        pltpu.VMEM((_BINS_PAD * _L,), jnp.float32),
        pltpu.VMEM((_BINS_PAD,), jnp.float32),
        pltpu.VMEM((4, _BINS_PAD), jnp.float32),
        pltpu.VMEM((_BINS_PAD,), jnp.float32),
        pltpu.VMEM((32,), jnp.float32),
        pltpu.VMEM((4, 32), jnp.float32),
        pltpu.VMEM_SHARED((24, _BINS_PAD), jnp.float32),
        pltpu.VMEM_SHARED((24, _BINS_PAD), jnp.float32),
        pltpu.VMEM_SHARED((4096,), jnp.float32),
        pltpu.SemaphoreType.DMA,
        pltpu.SemaphoreType.DMA,
        pltpu.SemaphoreType.DMA,
        pltpu.SemaphoreType.DMA,
    ],
)
def _dice_partials(pred_hbm, label_hbm, out_hbm, *rest):
    _body(pred_hbm, label_hbm, out_hbm, *rest)


def kernel(pred, label):
    part = _dice_partials(pred, label)
    return (part[0] + part[1])[:_C] * (1.0 / 8.0)


# speedup vs baseline: 1.2971x; 1.2971x over previous
import functools

import jax
import jax.numpy as jnp
from jax import lax
from jax.experimental import pallas as pl
from jax.experimental.pallas import tpu as pltpu
from jax.experimental.pallas import tpu_sc as plsc

_C = 19
_BINS = _C * _C
_BINS_PAD = 368
_L = 16
_PX_PER_TILE = 65536
_CH = 16384
_NCH = _PX_PER_TILE // _CH


def _body(pred_hbm, label_hbm, out_hbm,
          pb0, pb1, lb0, lb1, hist, merged, sh4, bhist, svec, sc4,
          shared_hist, shared_scores, shared_pad,
          sp0, sp1, sl0, sl1):
    c = lax.axis_index("c")
    s = lax.axis_index("s")
    batch = c * 4 + s // 4
    row0 = (s % 4) * (_PX_PER_TILE // 512)

    lane = lax.iota(jnp.int32, _L)
    lane16 = lane * 16
    ones = jnp.ones((_L,), jnp.float32)
    zeros = jnp.zeros((_L,), jnp.float32)

    @plsc.parallel_loop(0, _BINS_PAD * _L // 256, 1)
    def _(k):
        for u in range(16):
            hist[pl.ds(k * 256 + u * 16, _L)] = zeros

    pbufs = (pb0, pb1)
    lbufs = (lb0, lb1)
    psems = (sp0, sp1)
    lsems = (sl0, sl1)

    rows_per_ch = _CH // 512

    def start(ch):
        slot = ch % 2
        r = row0 + ch * rows_per_ch
        cp = pltpu.async_copy(pred_hbm.at[batch, 0, pl.ds(r, rows_per_ch), :],
                              pbufs[slot], psems[slot])
        cl = pltpu.async_copy(label_hbm.at[batch, 0, pl.ds(r, rows_per_ch), :],
                              lbufs[slot], lsems[slot])
        return cp, cl

    pending = {0: start(0)}
    for ch in range(_NCH):
        if ch + 1 < _NCH:
            pending[ch + 1] = start(ch + 1)
        cp, cl = pending.pop(ch)
        cp.wait()
        cl.wait()
        pb = pbufs[ch % 2]
        lb = lbufs[ch % 2]

        def make_chunk_body(pb, lb):
            def chunk_body(v):
                r = v >> 5
                col = (v & 31) * 16
                p = pb[r, pl.ds(col, _L)]
                l = lb[r, pl.ds(col, _L)]
                addr = (p * _C + l) * 16 + lane
                plsc.addupdate_scatter(hist, [addr], ones)
            return chunk_body
        plsc.parallel_loop(0, _CH // 16, 1, unroll=16)(make_chunk_body(pb, lb))

    @plsc.parallel_loop(0, _BINS_PAD // 16, 1)
    def _(k):
        acc = zeros
        for u in range(16):
            acc = acc + plsc.load_gather(hist, [k * 256 + lane16 + u])
        merged[pl.ds(k * 16, _L)] = acc

    pltpu.sync_copy(merged, shared_hist.at[s])
    plsc.subcore_barrier()

    @pl.when(s < 4)
    def _():
        pltpu.sync_copy(shared_hist.at[pl.ds(s * 4, 4)], sh4)

        def sum4_body(k, _):
            d = pl.ds(k * 16, _L)
            bhist[d] = (sh4[0, d] + sh4[1, d]) + (sh4[2, d] + sh4[3, d])
            return 0
        lax.fori_loop(0, _BINS_PAD // 16, sum4_body, 0)

        for half in range(2):
            cvec = jnp.minimum(half * 16 + lane, _C - 1)
            hi = plsc.load_gather(bhist, [cvec * (_C + 1)])
            hp = jnp.zeros((_L,), jnp.float32)
            hl = jnp.zeros((_L,), jnp.float32)
            for v in range(_C):
                hp = hp + plsc.load_gather(bhist, [cvec * _C + v])
                hl = hl + plsc.load_gather(bhist, [v * _C + cvec])
            merged[pl.ds(half * 16, _L)] = (2.0 * hi) / (hp + hl + 1e-10)

    pltpu.sync_copy(merged, shared_scores.at[s])
    plsc.subcore_barrier()

    @pl.when(s == 0)
    def _():
        pltpu.sync_copy(shared_scores.at[pl.ds(0, 4)], sh4)
        for half in range(2):
            d = pl.ds(half * 16, _L)
            svec[d] = ((sh4[0, d] + sh4[1, d]) + (sh4[2, d] + sh4[3, d])) * 0.125
        pltpu.sync_copy(svec, out_hbm.at[c])


@functools.partial(
    pl.kernel,
    out_type=jax.ShapeDtypeStruct((2, 32), jnp.float32),
    mesh=plsc.VectorSubcoreMesh(core_axis_name="c", subcore_axis_name="s"),
    compiler_params=pltpu.CompilerParams(needs_layout_passes=False),
    scratch_types=[
        pltpu.VMEM((_CH // 512, 512), jnp.int32),
        pltpu.VMEM((_CH // 512, 512), jnp.int32),
        pltpu.VMEM((_CH // 512, 512), jnp.int32),
        pltpu.VMEM((_CH // 512, 512), jnp.int32),
        pltpu.VMEM((_BINS_PAD * _L,), jnp.float32),
        pltpu.VMEM((_BINS_PAD,), jnp.float32),
        pltpu.VMEM((4, _BINS_PAD), jnp.float32),
        pltpu.VMEM((_BINS_PAD,), jnp.float32),
        pltpu.VMEM((32,), jnp.float32),
        pltpu.VMEM((4, 32), jnp.float32),
        pltpu.VMEM_SHARED((24, _BINS_PAD), jnp.float32),
        pltpu.VMEM_SHARED((24, _BINS_PAD), jnp.float32),
        pltpu.VMEM_SHARED((4096,), jnp.float32),
        pltpu.SemaphoreType.DMA,
        pltpu.SemaphoreType.DMA,
        pltpu.SemaphoreType.DMA,
        pltpu.SemaphoreType.DMA,
    ],
)
def _dice_partials(pred_hbm, label_hbm, out_hbm, *rest):
    _body(pred_hbm, label_hbm, out_hbm, *rest)


def kernel(pred, label):
    part = _dice_partials(pred, label)
    return (part[0] + part[1])[:_C]
